# baseline (speedup 1.0000x reference)
import functools

import jax
import jax.numpy as jnp
from jax import lax
from jax.experimental import pallas as pl
from jax.experimental.pallas import tpu as pltpu
from jax.experimental.pallas import tpu_sc as plsc

VOCAB = 100000
H = 768
B = 1024
S = 50
N = B * S
L = 16
NLANE = H // L
NW = 32
PER_W = N // NW
CHUNK = 40
NCHUNK = PER_W // CHUNK
EPS = 1e-5


def _rsqrt16(v):
    i = lax.bitcast_convert_type(v, jnp.int32)
    i = jnp.int32(0x5F3759DF) - (i >> 1)
    y = lax.bitcast_convert_type(i, jnp.float32)
    for _ in range(3):
        y = y * (1.5 - 0.5 * v * y * y)
    return y


def _lane_sum(x):
    dnums = lax.GatherDimensionNumbers(
        offset_dims=(), collapsed_slice_dims=(0,), start_index_map=(0,))
    for stg in (8, 4, 2, 1):
        perm = (jnp.arange(L, dtype=jnp.int32) ^ stg)[:, None]
        x = x + lax.gather(x, perm, dnums, slice_sizes=(1,),
                           mode=lax.GatherScatterMode.PROMISE_IN_BOUNDS)
    return x


def _row_stats(read):
    s = jnp.zeros((L,), jnp.float32)
    q = jnp.zeros((L,), jnp.float32)
    for j in range(NLANE):
        x = read(j)
        s = s + x
        q = q + x * x
    m_vec = _lane_sum(s) * (1.0 / H)
    v_vec = _lane_sum(q) * (1.0 / H) - m_vec * m_vec
    inv_vec = _rsqrt16(v_vec + EPS)
    return m_vec, inv_vec


def _sc_kernel(table_hbm, idx_hbm, pos_hbm, gpos_hbm, bpos_hbm, gans_hbm,
               bans_hbm, out_hbm, idx_v, rows_v, padd_v, gans_v, gpos_v,
               bpos_v, bans_v, sem):
    wid = lax.axis_index("s") * 2 + lax.axis_index("c")
    base = wid * PER_W

    pltpu.sync_copy(idx_hbm.at[pl.ds(base, PER_W)], idx_v)
    pltpu.sync_copy(pos_hbm, padd_v)
    pltpu.sync_copy(gans_hbm, gans_v)
    pltpu.sync_copy(gpos_hbm, gpos_v)
    pltpu.sync_copy(bpos_hbm, bpos_v)
    pltpu.sync_copy(bans_hbm, bans_v)

    def pos_body(r, _):
        def read(j):
            return padd_v[r, pl.ds(j * L, L)]
        m, inv = _row_stats(read)
        for j in range(NLANE):
            sl = pl.ds(j * L, L)
            x = padd_v[r, sl]
            padd_v[r, sl] = ((x - m) * inv * gpos_v[sl]
                             + bpos_v[sl] + bans_v[sl])
        return _
    lax.fori_loop(0, S, pos_body, None)

    def chunk_body(t, _):
        idx_slice = idx_v.at[pl.ds(t * CHUNK, CHUNK)]
        pltpu.async_copy(table_hbm.at[idx_slice], rows_v, sem).wait()

        def row_body(r, _):
            p = lax.rem(t * CHUNK + r, S)
            def read(j):
                return rows_v[r, pl.ds(j * L, L)]
            m, inv = _row_stats(read)
            for j in range(NLANE):
                sl = pl.ds(j * L, L)
                x = rows_v[r, sl]
                rows_v[r, sl] = (x - m) * inv * gans_v[sl] + padd_v[p, sl]
            return _
        lax.fori_loop(0, CHUNK, row_body, None)

        pltpu.sync_copy(rows_v, out_hbm.at[pl.ds(base + t * CHUNK, CHUNK)])
        return _
    lax.fori_loop(0, NCHUNK, chunk_body, None)


@jax.jit
def _prev_embedding(fixed_ans_emb, idx, pos_table, ln_pos_g, ln_pos_b,
                    ln_ans_g, ln_ans_b):
    mesh = plsc.VectorSubcoreMesh(core_axis_name="c", subcore_axis_name="s")
    fn = pl.kernel(
        _sc_kernel,
        mesh=mesh,
        out_type=jax.ShapeDtypeStruct((N, H), jnp.float32),
        scratch_types=[
            pltpu.VMEM((PER_W,), jnp.int32),
            pltpu.VMEM((CHUNK, H), jnp.float32),
            pltpu.VMEM((S, H), jnp.float32),
            pltpu.VMEM((H,), jnp.float32),
            pltpu.VMEM((H,), jnp.float32),
            pltpu.VMEM((H,), jnp.float32),
            pltpu.VMEM((H,), jnp.float32),
            pltpu.SemaphoreType.DMA,
        ],
    )
    return fn(fixed_ans_emb, idx, pos_table, ln_pos_g, ln_pos_b,
              ln_ans_g, ln_ans_b)


def kernel(fixed_ans_emb, prev_inds, pos_table, ln_pos_g, ln_pos_b,
           ln_ans_g, ln_ans_b):
    idx = prev_inds.reshape(-1).astype(jnp.int32)
    out = _prev_embedding(fixed_ans_emb, idx, pos_table, ln_pos_g, ln_pos_b,
                          ln_ans_g, ln_ans_b)
    return out.reshape(B, S, H)


# speedup vs baseline: 1.8893x; 1.8893x over previous
import functools

import jax
import jax.numpy as jnp
from jax import lax
from jax.experimental import pallas as pl
from jax.experimental.pallas import tpu as pltpu
from jax.experimental.pallas import tpu_sc as plsc

VOCAB = 100000
H = 768
B = 1024
S = 50
N = B * S
NW = 32
PER_W = N // NW
CH = 40
NCH = PER_W // CH
DEPTH = 4
EPS = 1e-5
TCB = 8


def _sc_gather(table_hbm, idx_hbm, out_hbm, idx_v, bufs, sems):
    wid = lax.axis_index("s") * 2 + lax.axis_index("c")
    base = wid * PER_W
    pltpu.sync_copy(idx_hbm.at[pl.ds(base, PER_W)], idx_v)

    def g_start(t, b):
        src = table_hbm.at[idx_v.at[pl.ds(t * CH, CH)]]
        pltpu.async_copy(src, bufs[b], sems[b])

    def g_wait(t, b):
        src = table_hbm.at[idx_v.at[pl.ds(t * CH, CH)]]
        pltpu.make_async_copy(src, bufs[b], sems[b]).wait()

    for b in range(DEPTH):
        g_start(b, b)

    def outer(g, _):
        for b in range(DEPTH):
            t = g * DEPTH + b
            g_wait(t, b)
            pltpu.sync_copy(bufs[b], out_hbm.at[pl.ds(base + t * CH, CH)])
            nxt = t + DEPTH

            @pl.when(nxt < NCH)
            def _():
                g_start(nxt, b)
        return _
    lax.fori_loop(0, NCH // DEPTH, outer, None)


@jax.jit
def _gather_rows(fixed_ans_emb, idx):
    mesh = plsc.VectorSubcoreMesh(core_axis_name="c", subcore_axis_name="s")
    fn = pl.kernel(
        _sc_gather,
        mesh=mesh,
        out_type=jax.ShapeDtypeStruct((N, H), jnp.float32),
        scratch_types=[
            pltpu.VMEM((PER_W,), jnp.int32),
            [pltpu.VMEM((CH, H), jnp.float32) for _ in range(DEPTH)],
            [pltpu.SemaphoreType.DMA for _ in range(DEPTH)],
        ],
    )
    return fn(fixed_ans_emb, idx)


def _tc_ln(gath_ref, pos_ref, gpos_ref, bpos_ref, gans_ref, bans_ref,
           out_ref, padd_ref):
    @pl.when(pl.program_id(0) == 0)
    def _():
        p = pos_ref[...]
        m = jnp.mean(p, axis=1, keepdims=True)
        d = p - m
        v = jnp.mean(d * d, axis=1, keepdims=True)
        padd_ref[...] = (d * lax.rsqrt(v + EPS) * gpos_ref[...]
                         + bpos_ref[...] + bans_ref[...])

    x = gath_ref[...]
    m = jnp.mean(x, axis=2, keepdims=True)
    d = x - m
    v = jnp.mean(d * d, axis=2, keepdims=True)
    out_ref[...] = (d * lax.rsqrt(v + EPS) * gans_ref[...][None]
                    + padd_ref[...][None])


@jax.jit
def _ln_posadd(gathered, pos_table, ln_pos_g, ln_pos_b, ln_ans_g, ln_ans_b):
    g3 = gathered.reshape(B, S, H)
    return pl.pallas_call(
        _tc_ln,
        grid=(B // TCB,),
        in_specs=[
            pl.BlockSpec((TCB, S, H), lambda i: (i, 0, 0)),
            pl.BlockSpec((S, H), lambda i: (0, 0)),
            pl.BlockSpec((1, H), lambda i: (0, 0)),
            pl.BlockSpec((1, H), lambda i: (0, 0)),
            pl.BlockSpec((1, H), lambda i: (0, 0)),
            pl.BlockSpec((1, H), lambda i: (0, 0)),
        ],
        out_specs=pl.BlockSpec((TCB, S, H), lambda i: (i, 0, 0)),
        out_shape=jax.ShapeDtypeStruct((B, S, H), jnp.float32),
        scratch_shapes=[pltpu.VMEM((S, H), jnp.float32)],
    )(g3, pos_table, ln_pos_g.reshape(1, H), ln_pos_b.reshape(1, H),
      ln_ans_g.reshape(1, H), ln_ans_b.reshape(1, H))


def kernel(fixed_ans_emb, prev_inds, pos_table, ln_pos_g, ln_pos_b,
           ln_ans_g, ln_ans_b):
    idx = prev_inds.reshape(-1).astype(jnp.int32)
    gathered = _gather_rows(fixed_ans_emb, idx)
    return _ln_posadd(gathered, pos_table, ln_pos_g, ln_pos_b,
                      ln_ans_g, ln_ans_b)


# speedup vs baseline: 2.5657x; 1.3580x over previous
import functools

import jax
import jax.numpy as jnp
from jax import lax
from jax.experimental import pallas as pl
from jax.experimental.pallas import tpu as pltpu
from jax.experimental.pallas import tpu_sc as plsc

VOCAB = 100000
H = 768
B = 1024
S = 50
SA = 48
SB = 2
SP = 56
NW = 32
BPW = B // NW
DEPTH = 2
EPS = 1e-5
TCB = 8


def _sc_gather(table_hbm, idx_hbm, out_hbm, idx_v, bufs_a, bufs_b, sems):
    wid = lax.axis_index("s") * 2 + lax.axis_index("c")
    base = wid * BPW
    pltpu.sync_copy(idx_hbm.at[pl.ds(base * SP, BPW * SP)], idx_v)

    def g_start(t, b):
        pltpu.async_copy(table_hbm.at[idx_v.at[pl.ds(t * SP, SA)]],
                         bufs_a[b], sems[b])
        pltpu.async_copy(table_hbm.at[idx_v.at[pl.ds(t * SP + SA, SB)]],
                         bufs_b[b], sems[b])

    def g_wait(t, b):
        pltpu.make_async_copy(table_hbm.at[idx_v.at[pl.ds(t * SP, SA)]],
                              bufs_a[b], sems[b]).wait()
        pltpu.make_async_copy(table_hbm.at[idx_v.at[pl.ds(t * SP + SA, SB)]],
                              bufs_b[b], sems[b]).wait()

    for b in range(DEPTH):
        g_start(b, b)

    def outer(g, _):
        for b in range(DEPTH):
            t = g * DEPTH + b
            g_wait(t, b)
            pltpu.sync_copy(bufs_a[b], out_hbm.at[base + t, pl.ds(0, SA)])
            pltpu.sync_copy(bufs_b[b], out_hbm.at[base + t, pl.ds(SA, SB)])
            nxt = t + DEPTH

            @pl.when(nxt < BPW)
            def _():
                g_start(nxt, b)
        return _
    lax.fori_loop(0, BPW // DEPTH, outer, None)


@jax.jit
def _gather_rows(fixed_ans_emb, idx_arranged):
    mesh = plsc.VectorSubcoreMesh(core_axis_name="c", subcore_axis_name="s")
    fn = pl.kernel(
        _sc_gather,
        mesh=mesh,
        out_type=jax.ShapeDtypeStruct((B, S, H), jnp.float32),
        scratch_types=[
            pltpu.VMEM((BPW * SP,), jnp.int32),
            [pltpu.VMEM((SA, H), jnp.float32) for _ in range(DEPTH)],
            [pltpu.VMEM((SB, H), jnp.float32) for _ in range(DEPTH)],
            [pltpu.SemaphoreType.DMA for _ in range(DEPTH)],
        ],
    )
    return fn(fixed_ans_emb, idx_arranged)


def _tc_ln(gath_ref, pos_ref, gpos_ref, bpos_ref, gans_ref, bans_ref,
           out_ref, padd_ref):
    @pl.when(pl.program_id(0) == 0)
    def _():
        p = pos_ref[...]
        m = jnp.mean(p, axis=1, keepdims=True)
        d = p - m
        v = jnp.mean(d * d, axis=1, keepdims=True)
        padd_ref[...] = (d * lax.rsqrt(v + EPS) * gpos_ref[...]
                         + bpos_ref[...] + bans_ref[...])

    x = gath_ref[...]
    m = jnp.mean(x, axis=2, keepdims=True)
    v = jnp.mean(x * x, axis=2, keepdims=True) - m * m
    out_ref[...] = ((x - m) * lax.rsqrt(v + EPS) * gans_ref[...][None]
                    + padd_ref[...][None])


@jax.jit
def _ln_posadd(gathered, pos_table, ln_pos_g, ln_pos_b, ln_ans_g, ln_ans_b):
    return pl.pallas_call(
        _tc_ln,
        grid=(B // TCB,),
        in_specs=[
            pl.BlockSpec((TCB, S, H), lambda i: (i, 0, 0)),
            pl.BlockSpec((S, H), lambda i: (0, 0)),
            pl.BlockSpec((1, H), lambda i: (0, 0)),
            pl.BlockSpec((1, H), lambda i: (0, 0)),
            pl.BlockSpec((1, H), lambda i: (0, 0)),
            pl.BlockSpec((1, H), lambda i: (0, 0)),
        ],
        out_specs=pl.BlockSpec((TCB, S, H), lambda i: (i, 0, 0)),
        out_shape=jax.ShapeDtypeStruct((B, S, H), jnp.float32),
        scratch_shapes=[pltpu.VMEM((S, H), jnp.float32)],
    )(gathered, pos_table, ln_pos_g.reshape(1, H), ln_pos_b.reshape(1, H),
      ln_ans_g.reshape(1, H), ln_ans_b.reshape(1, H))


def kernel(fixed_ans_emb, prev_inds, pos_table, ln_pos_g, ln_pos_b,
           ln_ans_g, ln_ans_b):
    idx = prev_inds.astype(jnp.int32)
    idx_arranged = jnp.pad(idx, ((0, 0), (0, SP - S))).reshape(-1)
    gathered = _gather_rows(fixed_ans_emb, idx_arranged)
    return _ln_posadd(gathered, pos_table, ln_pos_g, ln_pos_b,
                      ln_ans_g, ln_ans_b)
